# baseline (speedup 1.0000x reference)
import functools

import numpy as np
import jax
import jax.numpy as jnp
from jax import lax
from jax.experimental import pallas as pl
from jax.experimental.pallas import tpu as pltpu
from jax.experimental.pallas import tpu_sc as plsc

N = 10000
E = 320000
D = 128
NC = 2
NS = 16
NW = NC * NS
EPT = E // NW
C = 40
CHUNKS = EPT // C
BLKC = 50
NBLK = CHUNKS // BLKC
NPAD = 10240
RPT = NPAD // NS
INV_SQRT_D = float(1.0 / np.sqrt(D))


def _edge_body(q_hbm, k_hbm, v_hbm, src_hbm, dst_hbm,
               agg_out, den_out,
               qb0, kb0, vb0, vbs0, sem0q, sem0k, sem0v, sem0s,
               qb1, kb1, vb1, vbs1, sem1q, sem1k, sem1v, sem1s,
               sids, dids, dstrow0, dstrow1,
               spart, exb, den_v, agg_sh):
    c = lax.axis_index("c")
    s = lax.axis_index("s")
    wid = s * NC + c
    zero16 = jnp.zeros((16,), jnp.float32)
    bufs = ((qb0, kb0, vb0, vbs0, sem0q, sem0k, sem0v, sem0s, dstrow0),
            (qb1, kb1, vb1, vbs1, sem1q, sem1k, sem1v, sem1s, dstrow1))

    def _zden(i, carry):
        den_v[pl.ds(i * 16, 16)] = zero16
        return carry
    lax.fori_loop(0, NPAD // 16, _zden, 0)

    def _zvb(e, carry):
        for f in range(8):
            vbs0[e, pl.ds(16 * f, 16)] = zero16
            vbs1[e, pl.ds(16 * f, 16)] = zero16
        return carry
    lax.fori_loop(0, C, _zvb, 0)
    iota16 = lax.iota(jnp.int32, 16)
    base = s * RPT
    for r in range(RPT // C):
        pltpu.sync_copy(vbs0, agg_sh.at[pl.ds(base + r * C, C)])
    plsc.subcore_barrier()
    for _dr in (dstrow0, dstrow1):
        _dr[0, pl.ds(0, 16)] = iota16
        _dr[0, pl.ds(16, 16)] = iota16 + 16
        _dr[0, pl.ds(24, 16)] = iota16 + 24
    pltpu.async_copy(vbs0, agg_sh.at[dstrow0.at[0]], sem0s, add=True)
    pltpu.async_copy(vbs1, agg_sh.at[dstrow1.at[0]], sem1s, add=True)

    def _issue(row, b):
        qb, kb, vb, vbs, sq, sk, sv, ss, dstrow = bufs[b]
        pltpu.async_copy(q_hbm.at[dids.at[pl.ds(row * C, C)]], qb, sq)
        pltpu.async_copy(k_hbm.at[sids.at[pl.ds(row * C, C)]], kb, sk)
        pltpu.async_copy(v_hbm.at[sids.at[pl.ds(row * C, C)]], vb, sv)

    def _compute(row, b, more, nxt):
        qb, kb, vb, vbs, sq, sk, sv, ss, dstrow = bufs[b]
        pltpu.make_async_copy(q_hbm.at[dids.at[pl.ds(row * C, C)]],
                              qb, sq).wait()
        pltpu.make_async_copy(k_hbm.at[sids.at[pl.ds(row * C, C)]],
                              kb, sk).wait()
        pltpu.make_async_copy(v_hbm.at[sids.at[pl.ds(row * C, C)]],
                              vb, sv).wait()
        pltpu.make_async_copy(vbs, agg_sh.at[dstrow.at[0]], ss).wait()

        def _dot(e, carry2):
            acc = None
            for f in range(4):
                qv = qb[e, pl.ds(32 * f, 32)]
                kv = kb[e, pl.ds(32 * f, 32)]
                qa, qc = plsc.unpack(qv, format=plsc.PackFormat.INTERLEAVED)
                ka, kc = plsc.unpack(kv, format=plsc.PackFormat.INTERLEAVED)
                term = qa * ka + qc * kc
                acc = term if acc is None else acc + term
            spart[pl.ds(e * 16, 16)] = acc
            return carry2
        lax.fori_loop(0, C, _dot, 0)

        for off, mask in ((0, None), (16, None), (24, iota16 >= 8)):
            colbase = off * 16 + iota16 * 16
            tot = plsc.load_gather(spart, [colbase])
            for cc in range(1, 16):
                tot = tot + plsc.load_gather(spart, [colbase + cc])
            ex = jnp.exp(tot * INV_SQRT_D)
            dstv = dids[pl.ds(row * C + off, 16)]
            if mask is None:
                plsc.addupdate_scatter(den_v, [dstv], ex)
            else:
                plsc.addupdate_scatter(den_v, [dstv], ex, mask=mask)
            exb[pl.ds(off, 16)] = ex
            dstrow[0, pl.ds(off, 16)] = dstv

        @pl.when(more)
        def _():
            pltpu.async_copy(q_hbm.at[dids.at[pl.ds(nxt * C, C)]], qb, sq)
            pltpu.async_copy(k_hbm.at[sids.at[pl.ds(nxt * C, C)]], kb, sk)

        def _scale(e, carry2):
            sv2 = plsc.load_gather(exb, [jnp.full((16,), e, jnp.int32)])
            for f in range(8):
                vbs[e, pl.ds(16 * f, 16)] = vb[e, pl.ds(16 * f, 16)] * sv2
            return carry2
        lax.fori_loop(0, C, _scale, 0)

        @pl.when(more)
        def _():
            pltpu.async_copy(v_hbm.at[sids.at[pl.ds(nxt * C, C)]], vb, sv)

        pltpu.async_copy(vbs, agg_sh.at[dstrow.at[0]], ss, add=True)

    def _blk(blk, carry):
        pltpu.sync_copy(src_hbm.at[wid, blk], sids)
        pltpu.sync_copy(dst_hbm.at[wid, blk], dids)
        _issue(0, 0)
        _issue(1, 1)

        def _pair(t, carry2):
            j = 2 * t
            more = t < BLKC // 2 - 1
            _compute(j, 0, more, j + 2)
            _compute(j + 1, 1, more, j + 3)
            return carry2
        lax.fori_loop(0, BLKC // 2, _pair, 0)
        return carry
    lax.fori_loop(0, NBLK, _blk, 0)

    pltpu.make_async_copy(vbs0, agg_sh.at[dstrow0.at[0]], sem0s).wait()
    pltpu.make_async_copy(vbs1, agg_sh.at[dstrow1.at[0]], sem1s).wait()
    plsc.subcore_barrier()
    pltpu.sync_copy(den_v, den_out.at[wid])
    pltpu.sync_copy(agg_sh.at[pl.ds(base, RPT)],
                    agg_out.at[c, pl.ds(base, RPT)])


_edge_pass = functools.partial(
    pl.kernel,
    out_type=(
        jax.ShapeDtypeStruct((NC, NPAD, D), jnp.float32),
        jax.ShapeDtypeStruct((NW, NPAD), jnp.float32),
    ),
    mesh=plsc.VectorSubcoreMesh(core_axis_name="c", subcore_axis_name="s",
                                num_cores=NC, num_subcores=NS),
    compiler_params=pltpu.CompilerParams(needs_layout_passes=False,
                                         use_tc_tiling_on_sc=False),
    scratch_types=[
        pltpu.VMEM((C, D), jnp.bfloat16),
        pltpu.VMEM((C, D), jnp.bfloat16),
        pltpu.VMEM((C, D), jnp.float32),
        pltpu.VMEM((C, D), jnp.float32),
        pltpu.SemaphoreType.DMA,
        pltpu.SemaphoreType.DMA,
        pltpu.SemaphoreType.DMA,
        pltpu.SemaphoreType.DMA,
        pltpu.VMEM((C, D), jnp.bfloat16),
        pltpu.VMEM((C, D), jnp.bfloat16),
        pltpu.VMEM((C, D), jnp.float32),
        pltpu.VMEM((C, D), jnp.float32),
        pltpu.SemaphoreType.DMA,
        pltpu.SemaphoreType.DMA,
        pltpu.SemaphoreType.DMA,
        pltpu.SemaphoreType.DMA,
        pltpu.VMEM((BLKC * C,), jnp.int32),
        pltpu.VMEM((BLKC * C,), jnp.int32),
        pltpu.VMEM((8, C), jnp.int32),
        pltpu.VMEM((8, C), jnp.int32),
        pltpu.VMEM((C * 16,), jnp.float32),
        pltpu.VMEM((C,), jnp.float32),
        pltpu.VMEM((NPAD,), jnp.float32),
        pltpu.VMEM_SHARED((NPAD, D), jnp.float32),
    ],
)(_edge_body)


def _proj_body(x_ref, w_ref, b_ref, o_ref):
    o_ref[...] = (
        jnp.dot(x_ref[...], w_ref[...], preferred_element_type=jnp.float32)
        + b_ref[...]
    )


def _proj(x, wcat, bcat):
    n, din = x.shape
    dout = wcat.shape[1]
    blk = 2000
    return pl.pallas_call(
        _proj_body,
        grid=(n // blk,),
        in_specs=[
            pl.BlockSpec((blk, din), lambda i: (i, 0)),
            pl.BlockSpec((din, dout), lambda i: (0, 0)),
            pl.BlockSpec((1, dout), lambda i: (0, 0)),
        ],
        out_specs=pl.BlockSpec((blk, dout), lambda i: (i, 0)),
        out_shape=jax.ShapeDtypeStruct((n, dout), jnp.float32),
    )(x, wcat, bcat.reshape(1, dout))


def _combine_body(agg_ref, den_ref, skip_ref, o_ref, *, relu):
    den = jnp.sum(den_ref[...], axis=1, keepdims=True)
    agg = agg_ref[0, :N, :] + agg_ref[1, :N, :]
    r = agg / (den + 1e-16) + skip_ref[...]
    if relu:
        r = jnp.maximum(r, 0.0)
    o_ref[...] = r


def _combine(agg, den_t, skip, relu):
    return pl.pallas_call(
        functools.partial(_combine_body, relu=relu),
        in_specs=[
            pl.BlockSpec((NC, NPAD, D), lambda: (0, 0, 0)),
            pl.BlockSpec((N, NW), lambda: (0, 0)),
            pl.BlockSpec((N, D), lambda: (0, 0)),
        ],
        out_specs=pl.BlockSpec((N, D), lambda: (0, 0)),
        out_shape=jax.ShapeDtypeStruct((N, D), jnp.float32),
    )(agg, den_t, skip)


def _layer(x, src32, dst32, wcat, bcat, relu):
    p = _proj(x, wcat, bcat)
    q, k, v, s = (p[:, 0:D], p[:, D:2 * D], p[:, 2 * D:3 * D], p[:, 3 * D:])
    agg, den = _edge_pass(q.astype(jnp.bfloat16), k.astype(jnp.bfloat16),
                          v, src32, dst32)
    den_t = den[:, :N].T
    return _combine(agg, den_t, s, relu)


def kernel(x, edge_index, Wq1, bq1, Wk1, bk1, Wv1, bv1, Ws1, bs1,
           Wq2, bq2, Wk2, bk2, Wv2, bv2, Ws2, bs2):
    src32 = edge_index[0].reshape(NW, NBLK, BLKC * C)
    dst32 = edge_index[1].reshape(NW, NBLK, BLKC * C)
    wc1 = jnp.concatenate([Wq1, Wk1, Wv1, Ws1], axis=1)
    bc1 = jnp.concatenate([bq1, bk1, bv1, bs1], axis=0)
    wc2 = jnp.concatenate([Wq2, Wk2, Wv2, Ws2], axis=1)
    bc2 = jnp.concatenate([bq2, bk2, bv2, bs2], axis=0)
    h = _layer(x, src32, dst32, wc1, bc1, relu=True)
    return _layer(h, src32, dst32, wc2, bc2, relu=False)


# speedup vs baseline: 1.5293x; 1.5293x over previous
import functools

import numpy as np
import jax
import jax.numpy as jnp
from jax import lax
from jax.experimental import pallas as pl
from jax.experimental.pallas import tpu as pltpu
from jax.experimental.pallas import tpu_sc as plsc

N = 10000
E = 320000
D = 128
NC = 2
NS = 16
NW = NC * NS
EPT = E // NW
C = 80
CHUNKS = EPT // C
BLKC = 25
NBLK = CHUNKS // BLKC
NPAD = 10240
RPT = NPAD // NS
INV_SQRT_D = float(1.0 / np.sqrt(D))


def _edge_body(q_hbm, k_hbm, v_hbm, src_hbm, dst_hbm,
               agg_out, den_out,
               qb0, kb0, sem0q, sem0k,
               qb1, kb1, sem1q, sem1k,
               vb, semv, sids, dids, dstrow,
               spart, exb, den_v, agg_sh):
    c = lax.axis_index("c")
    s = lax.axis_index("s")
    wid = s * NC + c
    zero16 = jnp.zeros((16,), jnp.float32)
    bufs = ((qb0, kb0, sem0q, sem0k), (qb1, kb1, sem1q, sem1k))

    def _zden(i, carry):
        den_v[pl.ds(i * 16, 16)] = zero16
        return carry
    lax.fori_loop(0, NPAD // 16, _zden, 0)

    def _zvb(e, carry):
        for f in range(8):
            vb[e, pl.ds(16 * f, 16)] = zero16
        return carry
    lax.fori_loop(0, C, _zvb, 0)
    base = s * RPT
    for r in range(RPT // C):
        pltpu.sync_copy(vb, agg_sh.at[pl.ds(base + r * C, C)])
    plsc.subcore_barrier()

    iota16 = lax.iota(jnp.int32, 16)

    def _issue_qk(row, b):
        qb, kb, sq, sk = bufs[b]
        pltpu.async_copy(q_hbm.at[dids.at[pl.ds(row * C, C)]], qb, sq)
        pltpu.async_copy(k_hbm.at[sids.at[pl.ds(row * C, C)]], kb, sk)

    def _issue_v(row):
        pltpu.async_copy(v_hbm.at[sids.at[pl.ds(row * C, C)]], vb, semv)

    def _compute(row, b, more_qk, more_v):
        qb, kb, sq, sk = bufs[b]
        pltpu.make_async_copy(q_hbm.at[dids.at[pl.ds(row * C, C)]],
                              qb, sq).wait()
        pltpu.make_async_copy(k_hbm.at[sids.at[pl.ds(row * C, C)]],
                              kb, sk).wait()

        def _dot(e, carry2):
            acc = None
            for f in range(4):
                qv = qb[e, pl.ds(32 * f, 32)]
                kv = kb[e, pl.ds(32 * f, 32)]
                qa, qc = plsc.unpack(qv, format=plsc.PackFormat.INTERLEAVED)
                ka, kc = plsc.unpack(kv, format=plsc.PackFormat.INTERLEAVED)
                term = qa * ka + qc * kc
                acc = term if acc is None else acc + term
            spart[pl.ds(e * 16, 16)] = acc
            return carry2
        lax.fori_loop(0, C, _dot, 0)

        for g in range(C // 16):
            off = 16 * g
            colbase = off * 16 + iota16 * 16
            tot = plsc.load_gather(spart, [colbase])
            for cc in range(1, 16):
                tot = tot + plsc.load_gather(spart, [colbase + cc])
            ex = jnp.exp(tot * INV_SQRT_D)
            dstv = dids[pl.ds(row * C + off, 16)]
            plsc.addupdate_scatter(den_v, [dstv], ex)
            exb[pl.ds(off, 16)] = ex
            dstrow[0, pl.ds(off, 16)] = dstv

        @pl.when(more_qk)
        def _():
            pltpu.async_copy(q_hbm.at[dids.at[pl.ds((row + 2) * C, C)]],
                             qb, sq)
            pltpu.async_copy(k_hbm.at[sids.at[pl.ds((row + 2) * C, C)]],
                             kb, sk)

        pltpu.make_async_copy(v_hbm.at[sids.at[pl.ds(row * C, C)]],
                              vb, semv).wait()

        def _scale(e, carry2):
            sv2 = plsc.load_gather(exb, [jnp.full((16,), e, jnp.int32)])
            for f in range(8):
                vb[e, pl.ds(16 * f, 16)] = vb[e, pl.ds(16 * f, 16)] * sv2
            return carry2
        lax.fori_loop(0, C, _scale, 0)

        pltpu.sync_copy(vb, agg_sh.at[dstrow.at[0]], add=True)

        @pl.when(more_v)
        def _():
            _issue_v(row + 1)

    def _blk(blk, carry):
        pltpu.sync_copy(src_hbm.at[wid, blk], sids)
        pltpu.sync_copy(dst_hbm.at[wid, blk], dids)
        _issue_qk(0, 0)
        _issue_qk(1, 1)
        _issue_v(0)

        def _pair(t, carry2):
            j = 2 * t
            _compute(j, 0, True, True)
            _compute(j + 1, 1, t < BLKC // 2 - 1, True)
            return carry2
        lax.fori_loop(0, BLKC // 2, _pair, 0)
        _compute(BLKC - 1, 0, False, False)
        return carry
    lax.fori_loop(0, NBLK, _blk, 0)

    plsc.subcore_barrier()
    pltpu.sync_copy(den_v, den_out.at[wid])
    pltpu.sync_copy(agg_sh.at[pl.ds(base, RPT)],
                    agg_out.at[c, pl.ds(base, RPT)])


_edge_pass = functools.partial(
    pl.kernel,
    out_type=(
        jax.ShapeDtypeStruct((NC, NPAD, D), jnp.float32),
        jax.ShapeDtypeStruct((NW, NPAD), jnp.float32),
    ),
    mesh=plsc.VectorSubcoreMesh(core_axis_name="c", subcore_axis_name="s",
                                num_cores=NC, num_subcores=NS),
    compiler_params=pltpu.CompilerParams(needs_layout_passes=False,
                                         use_tc_tiling_on_sc=False),
    scratch_types=[
        pltpu.VMEM((C, D), jnp.bfloat16),
        pltpu.VMEM((C, D), jnp.bfloat16),
        pltpu.SemaphoreType.DMA,
        pltpu.SemaphoreType.DMA,
        pltpu.VMEM((C, D), jnp.bfloat16),
        pltpu.VMEM((C, D), jnp.bfloat16),
        pltpu.SemaphoreType.DMA,
        pltpu.SemaphoreType.DMA,
        pltpu.VMEM((C, D), jnp.float32),
        pltpu.SemaphoreType.DMA,
        pltpu.VMEM((BLKC * C,), jnp.int32),
        pltpu.VMEM((BLKC * C,), jnp.int32),
        pltpu.VMEM((8, C), jnp.int32),
        pltpu.VMEM((C * 16,), jnp.float32),
        pltpu.VMEM((C,), jnp.float32),
        pltpu.VMEM((NPAD,), jnp.float32),
        pltpu.VMEM_SHARED((NPAD, D), jnp.float32),
    ],
)(_edge_body)


def _proj_body(x_ref, w_ref, b_ref, o_ref):
    o_ref[...] = (
        jnp.dot(x_ref[...], w_ref[...], preferred_element_type=jnp.float32)
        + b_ref[...]
    )


def _proj(x, wcat, bcat):
    n, din = x.shape
    dout = wcat.shape[1]
    blk = 2000
    return pl.pallas_call(
        _proj_body,
        grid=(n // blk,),
        in_specs=[
            pl.BlockSpec((blk, din), lambda i: (i, 0)),
            pl.BlockSpec((din, dout), lambda i: (0, 0)),
            pl.BlockSpec((1, dout), lambda i: (0, 0)),
        ],
        out_specs=pl.BlockSpec((blk, dout), lambda i: (i, 0)),
        out_shape=jax.ShapeDtypeStruct((n, dout), jnp.float32),
    )(x, wcat, bcat.reshape(1, dout))


def _combine_body(agg_ref, den_ref, skip_ref, o_ref, *, relu):
    den = jnp.sum(den_ref[...], axis=1, keepdims=True)
    agg = agg_ref[0, :N, :] + agg_ref[1, :N, :]
    r = agg / (den + 1e-16) + skip_ref[...]
    if relu:
        r = jnp.maximum(r, 0.0)
    o_ref[...] = r


def _combine(agg, den_t, skip, relu):
    return pl.pallas_call(
        functools.partial(_combine_body, relu=relu),
        in_specs=[
            pl.BlockSpec((NC, NPAD, D), lambda: (0, 0, 0)),
            pl.BlockSpec((N, NW), lambda: (0, 0)),
            pl.BlockSpec((N, D), lambda: (0, 0)),
        ],
        out_specs=pl.BlockSpec((N, D), lambda: (0, 0)),
        out_shape=jax.ShapeDtypeStruct((N, D), jnp.float32),
    )(agg, den_t, skip)


def _layer(x, src32, dst32, wcat, bcat, relu):
    p = _proj(x, wcat, bcat)
    q, k, v, s = (p[:, 0:D], p[:, D:2 * D], p[:, 2 * D:3 * D], p[:, 3 * D:])
    agg, den = _edge_pass(q.astype(jnp.bfloat16), k.astype(jnp.bfloat16),
                          v, src32, dst32)
    den_t = den[:, :N].T
    return _combine(agg, den_t, s, relu)


def kernel(x, edge_index, Wq1, bq1, Wk1, bk1, Wv1, bv1, Ws1, bs1,
           Wq2, bq2, Wk2, bk2, Wv2, bv2, Ws2, bs2):
    src32 = edge_index[0].reshape(NW, NBLK, BLKC * C)
    dst32 = edge_index[1].reshape(NW, NBLK, BLKC * C)
    wc1 = jnp.concatenate([Wq1, Wk1, Wv1, Ws1], axis=1)
    bc1 = jnp.concatenate([bq1, bk1, bv1, bs1], axis=0)
    wc2 = jnp.concatenate([Wq2, Wk2, Wv2, Ws2], axis=1)
    bc2 = jnp.concatenate([bq2, bk2, bv2, bs2], axis=0)
    h = _layer(x, src32, dst32, wc1, bc1, relu=True)
    return _layer(h, src32, dst32, wc2, bc2, relu=False)
